# baseline (speedup 1.0000x reference)
import functools

import jax
import jax.numpy as jnp
from jax import lax
from jax.experimental import pallas as pl
from jax.experimental.pallas import tpu as pltpu
from jax.experimental.pallas import tpu_sc as plsc

_N = 10000
_NP = 10240
_E = 320000
_F = 128
_G = 64
_NW = 32
_CHUNK = 128
_NCHUNK = 80
_EP = _NW * _NCHUNK * _CHUNK
_ROWS_PER_TILE = _NP // 16


def _sc_degree_body(idx_hbm, deg_out, idx_v, hist):
    c = lax.axis_index("c")
    s = lax.axis_index("s")
    wid = s * 2 + c
    zeros16 = jnp.zeros((16,), jnp.float32)

    def zinit(i, carry):
        hist[pl.ds(i * 16, 16)] = zeros16
        return carry

    lax.fori_loop(0, _NP // 16, zinit, 0)
    pltpu.sync_copy(idx_hbm.at[wid], idx_v)
    ones16 = jnp.ones((16,), jnp.float32)

    def body(j, carry):
        for k in range(_CHUNK // 16):
            idx = idx_v[j, 1, pl.ds(k * 16, 16)]
            plsc.addupdate_scatter(hist, [idx], ones16)
        return carry

    lax.fori_loop(0, _NCHUNK, body, 0)
    pltpu.sync_copy(hist, deg_out.at[wid])


def _sc_degree(idx4):
    mesh = plsc.VectorSubcoreMesh(core_axis_name="c", subcore_axis_name="s")
    fn = functools.partial(
        pl.kernel,
        out_type=jax.ShapeDtypeStruct((_NW, _NP), jnp.float32),
        mesh=mesh,
        scratch_types=[
            pltpu.VMEM((_NCHUNK, 2, _CHUNK), jnp.int32),
            pltpu.VMEM((_NP,), jnp.float32),
        ],
        compiler_params=pltpu.CompilerParams(needs_layout_passes=False),
    )(_sc_degree_body)
    return fn(idx4)


def _sc_agg_body(hs_hbm, idx_hbm, part_out, rows0, rows1, idx_v, acc,
                 si0, si1, si2, si3, sg0, sg1):
    c = lax.axis_index("c")
    s = lax.axis_index("s")
    wid = s * 2 + c
    rbuf = (rows0, rows1)
    sem_i = (si0, si1, si2, si3)
    sem_g = (sg0, sg1)
    zeros16 = jnp.zeros((16,), jnp.float32)

    def zrow(i, carry):
        for k in range(_F // 16):
            rows0[i, pl.ds(k * 16, 16)] = zeros16
        return carry

    lax.fori_loop(0, _CHUNK, zrow, 0)
    for b in range(_ROWS_PER_TILE // _CHUNK):
        pltpu.sync_copy(rows0, acc.at[pl.ds((s * 5 + b) * _CHUNK, _CHUNK)])
    plsc.subcore_barrier()

    def idx_wait(sl):
        pltpu.make_async_copy(idx_hbm.at[wid, 0], idx_v.at[sl], sem_i[sl]).wait()

    def gather_wait(rb):
        pltpu.make_async_copy(
            hs_hbm.at[pl.ds(0, _CHUNK)], rbuf[rb], sem_g[rb]).wait()

    for t in range(4):
        pltpu.async_copy(idx_hbm.at[wid, t], idx_v.at[t], sem_i[t])
    for t in range(2):
        idx_wait(t)
        pltpu.async_copy(hs_hbm.at[idx_v.at[t, 0]], rbuf[t], sem_g[t])

    def group(g, carry):
        jb = g * 4
        for u in range(4):
            j = jb + u
            rb = u % 2
            gather_wait(rb)
            pltpu.sync_copy(rbuf[rb], acc.at[idx_v.at[u, 1]], add=True)

            @pl.when(j + 4 < _NCHUNK)
            def _():
                pltpu.async_copy(idx_hbm.at[wid, j + 4], idx_v.at[u], sem_i[u])

            sl2 = (u + 2) % 4

            @pl.when(j + 2 < _NCHUNK)
            def _():
                idx_wait(sl2)
                pltpu.async_copy(hs_hbm.at[idx_v.at[sl2, 0]], rbuf[rb], sem_g[rb])
        return carry

    lax.fori_loop(0, _NCHUNK // 4, group, 0)
    plsc.subcore_barrier()
    pltpu.sync_copy(
        acc.at[pl.ds(s * _ROWS_PER_TILE, _ROWS_PER_TILE)],
        part_out.at[c, pl.ds(s * _ROWS_PER_TILE, _ROWS_PER_TILE)],
    )


def _sc_agg(hs, idx4):
    mesh = plsc.VectorSubcoreMesh(core_axis_name="c", subcore_axis_name="s")
    fn = functools.partial(
        pl.kernel,
        out_type=jax.ShapeDtypeStruct((2, _NP, _F), jnp.float32),
        mesh=mesh,
        scratch_types=[
            pltpu.VMEM((_CHUNK, _F), jnp.float32),
            pltpu.VMEM((_CHUNK, _F), jnp.float32),
            pltpu.VMEM((4, 2, _CHUNK), jnp.int32),
            pltpu.VMEM_SHARED((_NP, _F), jnp.float32),
            pltpu.SemaphoreType.DMA,
            pltpu.SemaphoreType.DMA,
            pltpu.SemaphoreType.DMA,
            pltpu.SemaphoreType.DMA,
            pltpu.SemaphoreType.DMA,
            pltpu.SemaphoreType.DMA,
        ],
        compiler_params=pltpu.CompilerParams(needs_layout_passes=False),
    )(_sc_agg_body)
    return fn(hs, idx4)


def _tc1_body(degp, x, w1, dinv_out, h1_out):
    deg = jnp.sum(degp[...], axis=0, keepdims=True) + 1.0
    dinv_out[...] = lax.rsqrt(deg)
    h1_out[...] = jnp.dot(x[...], w1[...], preferred_element_type=jnp.float32)


def _tc1(deg_parts, x, W1):
    return pl.pallas_call(
        _tc1_body,
        out_shape=(
            jax.ShapeDtypeStruct((1, _NP), jnp.float32),
            jax.ShapeDtypeStruct((_N, _F), jnp.float32),
        ),
    )(deg_parts, x, W1)


def _tc_scale_body(h, dinv, hs_out):
    hs_out[...] = h[...] * dinv[...]


def _tc_scale(h1, dinv_col):
    return pl.pallas_call(
        _tc_scale_body,
        out_shape=jax.ShapeDtypeStruct((_N, _F), jnp.float32),
    )(h1, dinv_col)


def _bn_relu(h, g, be):
    mu = jnp.mean(h, axis=0, keepdims=True)
    d = h - mu
    var = jnp.mean(d * d, axis=0, keepdims=True)
    return jnp.maximum(g * d * lax.rsqrt(var + 1e-5) + be, 0.0)


def _tc_mid_body(p, hs, dinv, b, g, be, w, hs_next_out):
    agg = p[0, : _N, :] + p[1, : _N, :] + hs[...]
    h = agg * dinv[...] + b[...]
    hr = _bn_relu(h, g[...], be[...])
    hs_next_out[...] = (
        jnp.dot(hr, w[...], preferred_element_type=jnp.float32) * dinv[...]
    )


def _tc_mid(p, hs, dinv_col, b, g, be, W):
    return pl.pallas_call(
        _tc_mid_body,
        out_shape=jax.ShapeDtypeStruct((_N, _F), jnp.float32),
    )(p, hs, dinv_col, b, g, be, W)


def _tc_final_body(p, hs, dinv, b, g, be, batch_t, l1w, l1b, l2w, l2b, out):
    agg = p[0, : _N, :] + p[1, : _N, :] + hs[...]
    h = agg * dinv[...] + b[...]
    hr = _bn_relu(h, g[...], be[...])
    ids = lax.broadcasted_iota(jnp.int32, (_G, _N), 0)
    onehot = (ids == batch_t[...]).astype(jnp.float32)
    cnt = jnp.sum(onehot, axis=1, keepdims=True)
    pooled = jnp.dot(onehot, hr, preferred_element_type=jnp.float32)
    pooled = pooled / jnp.maximum(cnt, 1.0)
    hm = jnp.maximum(
        jnp.dot(pooled, l1w[...], preferred_element_type=jnp.float32) + l1b[...],
        0.0,
    )
    out[...] = jnp.dot(hm, l2w[...], preferred_element_type=jnp.float32) + l2b[...]


def _tc_final(p, hs, dinv_col, b, g, be, batch_t, L1w, L1b, L2w, L2b):
    return pl.pallas_call(
        _tc_final_body,
        out_shape=jax.ShapeDtypeStruct((_G, _F), jnp.float32),
    )(p, hs, dinv_col, b, g, be, batch_t, L1w, L1b, L2w, L2b)


def kernel(x, edge_index, batch, W1, b1, g1, be1, W2, b2, g2, be2, W3, b3, g3,
           be3, L1w, L1b, L2w, L2b):
    pad = _EP - _E
    src3 = jnp.concatenate(
        [edge_index[0], jnp.zeros((pad,), jnp.int32)]).reshape(_NW, _NCHUNK, _CHUNK)
    dst3 = jnp.concatenate(
        [edge_index[1], jnp.full((pad,), _N, jnp.int32)]).reshape(_NW, _NCHUNK, _CHUNK)
    idx4 = jnp.stack([src3, dst3], axis=2)
    batch_t = batch.reshape(1, _N)
    b1r, g1r, be1r = b1.reshape(1, _F), g1.reshape(1, _F), be1.reshape(1, _F)
    b2r, g2r, be2r = b2.reshape(1, _F), g2.reshape(1, _F), be2.reshape(1, _F)
    b3r, g3r, be3r = b3.reshape(1, _F), g3.reshape(1, _F), be3.reshape(1, _F)
    l1br, l2br = L1b.reshape(1, _F), L2b.reshape(1, _F)

    deg_parts = _sc_degree(idx4)
    dinv_row, h1 = _tc1(deg_parts, x, W1)
    dinv_col = dinv_row.reshape(_NP, 1)[: _N]

    hs1 = _tc_scale(h1, dinv_col)
    p1 = _sc_agg(hs1, idx4)
    hs2 = _tc_mid(p1, hs1, dinv_col, b1r, g1r, be1r, W2)
    p2 = _sc_agg(hs2, idx4)
    hs3 = _tc_mid(p2, hs2, dinv_col, b2r, g2r, be2r, W3)
    p3 = _sc_agg(hs3, idx4)
    return _tc_final(p3, hs3, dinv_col, b3r, g3r, be3r, batch_t,
                     L1w, L1b, L2w, L2b)


# speedup vs baseline: 1.3316x; 1.3316x over previous
import functools

import jax
import jax.numpy as jnp
from jax import lax
from jax.experimental import pallas as pl
from jax.experimental.pallas import tpu as pltpu
from jax.experimental.pallas import tpu_sc as plsc

_N = 10000
_NP = 10240
_E = 320000
_F = 128
_G = 64
_NW = 32
_CHUNK = 128
_NCHUNK = 79
_EP = _NW * _NCHUNK * _CHUNK
_RPT = _NP // 16


def _sc_degree_body(dst_hbm, deg_out, dst_v, hist):
    c = lax.axis_index("c")
    s = lax.axis_index("s")
    wid = s * 2 + c
    zeros16 = jnp.zeros((16,), jnp.float32)

    def zinit(i, carry):
        hist[pl.ds(i * 16, 16)] = zeros16
        return carry

    lax.fori_loop(0, _NP // 16, zinit, 0)
    pltpu.sync_copy(dst_hbm.at[wid], dst_v)
    ones16 = jnp.ones((16,), jnp.float32)

    def body(j, carry):
        for k in range(_CHUNK // 16):
            idx = dst_v[j, pl.ds(k * 16, 16)]
            plsc.addupdate_scatter(hist, [idx], ones16)
        return carry

    lax.fori_loop(0, _NCHUNK, body, 0)
    pltpu.sync_copy(hist, deg_out.at[wid])


def _sc_degree(dst3):
    mesh = plsc.VectorSubcoreMesh(core_axis_name="c", subcore_axis_name="s")
    fn = functools.partial(
        pl.kernel,
        out_type=jax.ShapeDtypeStruct((_NW, _NP), jnp.float32),
        mesh=mesh,
        scratch_types=[
            pltpu.VMEM((_NCHUNK, _CHUNK), jnp.int32),
            pltpu.VMEM((_NP,), jnp.float32),
        ],
        compiler_params=pltpu.CompilerParams(needs_layout_passes=False),
    )(_sc_degree_body)
    return fn(dst3)


def _sc_agg_body(hs_hbm, src_hbm, dst_hbm, part_out, src_v, dst_v, rows, acc,
                 sem):
    c = lax.axis_index("c")
    s = lax.axis_index("s")
    wid = s * 2 + c
    zeros16 = jnp.zeros((16,), jnp.float32)

    def zrow(i, carry):
        for k in range(_F // 16):
            rows[i, pl.ds(k * 16, 16)] = zeros16
        return carry

    lax.fori_loop(0, _CHUNK, zrow, 0)
    for b in range(_RPT // _CHUNK):
        pltpu.sync_copy(rows, acc.at[pl.ds((s * 5 + b) * _CHUNK, _CHUNK)])
    plsc.subcore_barrier()

    pltpu.sync_copy(src_hbm.at[wid], src_v)
    pltpu.sync_copy(dst_hbm.at[wid], dst_v)

    def body(j, carry):
        pltpu.async_copy(hs_hbm.at[src_v.at[j]], rows, sem).wait()
        pltpu.sync_copy(rows, acc.at[dst_v.at[j]], add=True)
        return carry

    lax.fori_loop(0, _NCHUNK, body, 0)
    plsc.subcore_barrier()
    pltpu.sync_copy(
        acc.at[pl.ds(s * _RPT, _RPT)],
        part_out.at[c, pl.ds(s * _RPT, _RPT)],
    )


def _sc_agg(hs, src3, dst3):
    mesh = plsc.VectorSubcoreMesh(core_axis_name="c", subcore_axis_name="s")
    fn = functools.partial(
        pl.kernel,
        out_type=jax.ShapeDtypeStruct((2, _NP, _F), jnp.float32),
        mesh=mesh,
        scratch_types=[
            pltpu.VMEM((_NCHUNK, _CHUNK), jnp.int32),
            pltpu.VMEM((_NCHUNK, _CHUNK), jnp.int32),
            pltpu.VMEM((_CHUNK, _F), jnp.float32),
            pltpu.VMEM_SHARED((_NP, _F), jnp.float32),
            pltpu.SemaphoreType.DMA,
        ],
        compiler_params=pltpu.CompilerParams(needs_layout_passes=False),
    )(_sc_agg_body)
    return fn(hs, src3, dst3)


def _tc1_body(degp, x, w1, dinv_out, h1_out):
    deg = jnp.sum(degp[...], axis=0, keepdims=True) + 1.0
    dinv_out[...] = lax.rsqrt(deg)
    h1_out[...] = jnp.dot(x[...], w1[...], preferred_element_type=jnp.float32)


def _tc1(deg_parts, x, W1):
    return pl.pallas_call(
        _tc1_body,
        out_shape=(
            jax.ShapeDtypeStruct((1, _NP), jnp.float32),
            jax.ShapeDtypeStruct((_N, _F), jnp.float32),
        ),
    )(deg_parts, x, W1)


def _tc_scale_body(h, dinv, hs_out):
    hs_out[...] = h[...] * dinv[...]


def _tc_scale(h1, dinv_col):
    return pl.pallas_call(
        _tc_scale_body,
        out_shape=jax.ShapeDtypeStruct((_N, _F), jnp.float32),
    )(h1, dinv_col)


def _bn_relu(h, g, be):
    mu = jnp.mean(h, axis=0, keepdims=True)
    d = h - mu
    var = jnp.mean(d * d, axis=0, keepdims=True)
    return jnp.maximum(g * d * lax.rsqrt(var + 1e-5) + be, 0.0)


def _neigh_sum(p, hs):
    return p[0, : _N, :] + p[1, : _N, :] + hs[...]


def _tc_mid_body(p, hs, dinv, b, g, be, w, hs_next_out):
    h = _neigh_sum(p, hs) * dinv[...] + b[...]
    hr = _bn_relu(h, g[...], be[...])
    hs_next_out[...] = (
        jnp.dot(hr, w[...], preferred_element_type=jnp.float32) * dinv[...]
    )


def _tc_mid(p, hs, dinv_col, b, g, be, W):
    return pl.pallas_call(
        _tc_mid_body,
        out_shape=jax.ShapeDtypeStruct((_N, _F), jnp.float32),
    )(p, hs, dinv_col, b, g, be, W)


def _tc_final_body(p, hs, dinv, b, g, be, batch_t, l1w, l1b, l2w, l2b, out):
    h = _neigh_sum(p, hs) * dinv[...] + b[...]
    hr = _bn_relu(h, g[...], be[...])
    ids = lax.broadcasted_iota(jnp.int32, (_G, _N), 0)
    onehot = (ids == batch_t[...]).astype(jnp.float32)
    cnt = jnp.sum(onehot, axis=1, keepdims=True)
    pooled = jnp.dot(onehot, hr, preferred_element_type=jnp.float32)
    pooled = pooled / jnp.maximum(cnt, 1.0)
    hm = jnp.maximum(
        jnp.dot(pooled, l1w[...], preferred_element_type=jnp.float32) + l1b[...],
        0.0,
    )
    out[...] = jnp.dot(hm, l2w[...], preferred_element_type=jnp.float32) + l2b[...]


def _tc_final(p, hs, dinv_col, b, g, be, batch_t, L1w, L1b, L2w, L2b):
    return pl.pallas_call(
        _tc_final_body,
        out_shape=jax.ShapeDtypeStruct((_G, _F), jnp.float32),
    )(p, hs, dinv_col, b, g, be, batch_t, L1w, L1b, L2w, L2b)


def kernel(x, edge_index, batch, W1, b1, g1, be1, W2, b2, g2, be2, W3, b3, g3,
           be3, L1w, L1b, L2w, L2b):
    pad = _EP - _E
    src3 = jnp.concatenate(
        [edge_index[0], jnp.zeros((pad,), jnp.int32)]).reshape(_NW, _NCHUNK, _CHUNK)
    dst3 = jnp.concatenate(
        [edge_index[1], jnp.full((pad,), _N, jnp.int32)]).reshape(_NW, _NCHUNK, _CHUNK)
    batch_t = batch.reshape(1, _N)
    b1r, g1r, be1r = b1.reshape(1, _F), g1.reshape(1, _F), be1.reshape(1, _F)
    b2r, g2r, be2r = b2.reshape(1, _F), g2.reshape(1, _F), be2.reshape(1, _F)
    b3r, g3r, be3r = b3.reshape(1, _F), g3.reshape(1, _F), be3.reshape(1, _F)

    deg_parts = _sc_degree(dst3)
    dinv_row, h1 = _tc1(deg_parts, x, W1)
    dinv_col = dinv_row.reshape(_NP, 1)[: _N]

    hs1 = _tc_scale(h1, dinv_col)
    p1 = _sc_agg(hs1, src3, dst3)
    hs2 = _tc_mid(p1, hs1, dinv_col, b1r, g1r, be1r, W2)
    p2 = _sc_agg(hs2, src3, dst3)
    hs3 = _tc_mid(p2, hs2, dinv_col, b2r, g2r, be2r, W3)
    p3 = _sc_agg(hs3, src3, dst3)
    return _tc_final(p3, hs3, dinv_col, b3r, g3r, be3r, batch_t,
                     L1w, L1b, L2w, L2b)
